# baseline (device time: 43771 ns/iter reference)
import jax
import jax.numpy as jnp
from jax import lax
from jax.experimental import pallas as pl
from jax.experimental.pallas import tpu as pltpu

N_DEV = 32
B, SQ, D = 2, 256, 768
HQ_LOC, DH = 8, 64
KV_LOC = 2
ROWS = B * SQ
CHUNK = ROWS // N_DEV
BLK = 128
F = HQ_LOC * DH


def _body(x_ref, wq_ref, wk_ref, wv_ref, wo_ref, out_ref,
          qbuf, kbuf, vbuf, obuf, comm_ref,
          rs_send, rs_recv, ag_send, ag_recv):
    my = lax.axis_index("i")
    f32 = jnp.float32
    bf = jnp.bfloat16

    barrier = pltpu.get_barrier_semaphore()
    for k in range(1, N_DEV):
        t = lax.rem(my + k, N_DEV)
        pl.semaphore_signal(
            barrier, inc=1, device_id=(t,),
            device_id_type=pltpu.DeviceIdType.MESH,
        )
    pl.semaphore_wait(barrier, N_DEV - 1)

    x = x_ref[...]
    qbuf[...] = jax.lax.dot_general(
        x, wq_ref[...], (((1,), (0,)), ((), ())),
        preferred_element_type=f32).astype(bf)
    kbuf[...] = jax.lax.dot_general(
        x, wk_ref[...], (((1,), (0,)), ((), ())),
        preferred_element_type=f32).astype(bf)
    vbuf[...] = jax.lax.dot_general(
        x, wv_ref[...], (((1,), (0,)), ((), ())),
        preferred_element_type=f32).astype(bf)

    for b in range(B):
        rows = pl.ds(b * SQ, SQ)
        for h in range(HQ_LOC):
            g = h // (HQ_LOC // KV_LOC)
            q = qbuf[rows, pl.ds(h * DH, DH)]
            kk = kbuf[rows, pl.ds(g * DH, DH)]
            vv = vbuf[rows, pl.ds(g * DH, DH)]
            s = jax.lax.dot_general(
                q, kk, (((1,), (1,)), ((), ())),
                preferred_element_type=f32) * 0.125
            m = jnp.max(s, axis=1, keepdims=True)
            p = jnp.exp(s - m)
            l = jnp.sum(p, axis=1, keepdims=True)
            o = jax.lax.dot_general(
                p.astype(bf), vv, (((1,), (0,)), ((), ())),
                preferred_element_type=f32)
            obuf[rows, pl.ds(h * DH, DH)] = (o / l).astype(bf)

    for j in range(ROWS // BLK):
        rows = pl.ds(j * BLK, BLK)
        pb = jax.lax.dot_general(
            obuf[rows, :], wo_ref[...], (((1,), (0,)), ((), ())),
            preferred_element_type=f32)
        out_ref[rows, :] = pb.astype(bf)

    rs = []
    for k in range(1, N_DEV):
        t = lax.rem(my + k, N_DEV)
        rdma = pltpu.make_async_remote_copy(
            src_ref=out_ref.at[pl.ds(t * CHUNK, CHUNK), :],
            dst_ref=comm_ref.at[N_DEV - 1 - k],
            send_sem=rs_send.at[k - 1],
            recv_sem=rs_recv.at[N_DEV - 1 - k],
            device_id=(t,),
            device_id_type=pltpu.DeviceIdType.MESH,
        )
        rdma.start()
        rs.append(rdma)

    sl_my = pl.ds(my * CHUNK, CHUNK)
    acc = out_ref[sl_my, :].astype(f32)
    for h in range(N_DEV - 2, -1, -1):
        recv_desc = pltpu.make_async_remote_copy(
            src_ref=comm_ref.at[h],
            dst_ref=comm_ref.at[h],
            send_sem=rs_recv.at[0],
            recv_sem=rs_recv.at[h],
            device_id=(my,),
            device_id_type=pltpu.DeviceIdType.MESH,
        )
        recv_desc.wait_recv()
        acc = acc + comm_ref[h].astype(f32)
    out_ref[sl_my, :] = acc.astype(bf)

    ag = []
    for k in range(1, N_DEV):
        t = lax.rem(my + k, N_DEV)
        rdma = pltpu.make_async_remote_copy(
            src_ref=out_ref.at[sl_my, :],
            dst_ref=out_ref.at[sl_my, :],
            send_sem=ag_send.at[k - 1],
            recv_sem=ag_recv.at[N_DEV - 1 - k],
            device_id=(t,),
            device_id_type=pltpu.DeviceIdType.MESH,
        )
        rdma.start()
        ag.append(rdma)

    for h in range(N_DEV - 1):
        recv_desc = pltpu.make_async_remote_copy(
            src_ref=comm_ref.at[h],
            dst_ref=out_ref.at[pl.ds(0, CHUNK), :],
            send_sem=ag_send.at[0],
            recv_sem=ag_recv.at[h],
            device_id=(my,),
            device_id_type=pltpu.DeviceIdType.MESH,
        )
        recv_desc.wait_recv()
    for r in rs:
        r.wait_send()
    for r in ag:
        r.wait_send()


def kernel(x, Wq, Wo, Wk, Wv):
    i = lax.axis_index("i")
    bf = jnp.bfloat16

    x2 = x.reshape(ROWS, D).astype(bf)
    wk_loc = lax.dynamic_slice_in_dim(Wk, i * KV_LOC * DH, KV_LOC * DH, 1)
    wv_loc = lax.dynamic_slice_in_dim(Wv, i * KV_LOC * DH, KV_LOC * DH, 1)

    out = pl.pallas_call(
        _body,
        out_shape=jax.ShapeDtypeStruct((ROWS, D), bf),
        in_specs=[pl.BlockSpec(memory_space=pltpu.VMEM)] * 5,
        out_specs=pl.BlockSpec(memory_space=pltpu.VMEM),
        scratch_shapes=[
            pltpu.VMEM((ROWS, F), bf),
            pltpu.VMEM((ROWS, KV_LOC * DH), bf),
            pltpu.VMEM((ROWS, KV_LOC * DH), bf),
            pltpu.VMEM((ROWS, F), bf),
            pltpu.VMEM((N_DEV - 1, CHUNK, D), bf),
            pltpu.SemaphoreType.DMA((N_DEV - 1,)),
            pltpu.SemaphoreType.DMA((N_DEV - 1,)),
            pltpu.SemaphoreType.DMA((N_DEV - 1,)),
            pltpu.SemaphoreType.DMA((N_DEV - 1,)),
        ],
        compiler_params=pltpu.CompilerParams(collective_id=0),
    )(x2, Wq.astype(bf), wk_loc.astype(bf), wv_loc.astype(bf), Wo.astype(bf))
    return out.reshape(B, SQ, D)


# device time: 42055 ns/iter; 1.0408x vs baseline; 1.0408x over previous
import jax
import jax.numpy as jnp
from jax import lax
from jax.experimental import pallas as pl
from jax.experimental.pallas import tpu as pltpu

N_DEV = 32
B, SQ, D = 2, 256, 768
HQ_LOC, DH = 8, 64
KV_LOC = 2
ROWS = B * SQ
CHUNK = ROWS // N_DEV
BLK = 128
F = HQ_LOC * DH


def _body(x_ref, wq_ref, wk_ref, wv_ref, wo_ref, out_ref,
          qbuf, kbuf, vbuf, obuf, comm_ref,
          rs_send, rs_recv, ag_send, ag_recv):
    my = lax.axis_index("i")
    f32 = jnp.float32
    bf = jnp.bfloat16

    barrier = pltpu.get_barrier_semaphore()
    for k in range(1, N_DEV):
        t = lax.rem(my + k, N_DEV)
        pl.semaphore_signal(
            barrier, inc=1, device_id=(t,),
            device_id_type=pltpu.DeviceIdType.MESH,
        )

    x = x_ref[...]
    qbuf[...] = jax.lax.dot_general(
        wq_ref[...], x, (((0,), (1,)), ((), ())),
        preferred_element_type=f32).astype(bf)
    kbuf[...] = jax.lax.dot_general(
        wk_ref[...], x, (((0,), (1,)), ((), ())),
        preferred_element_type=f32).astype(bf)
    vbuf[...] = jax.lax.dot_general(
        wv_ref[...], x, (((0,), (1,)), ((), ())),
        preferred_element_type=f32).astype(bf)

    for b in range(B):
        cols = pl.ds(b * SQ, SQ)
        for h in range(HQ_LOC):
            g = h // (HQ_LOC // KV_LOC)
            qT = qbuf[pl.ds(h * DH, DH), cols]
            kT = kbuf[pl.ds(g * DH, DH), cols]
            vT = vbuf[pl.ds(g * DH, DH), cols]
            s = jax.lax.dot_general(
                qT, kT, (((0,), (0,)), ((), ())),
                preferred_element_type=f32) * 0.125
            m = jnp.max(s, axis=1, keepdims=True)
            p = jnp.exp(s - m)
            l = jnp.sum(p, axis=1, keepdims=True)
            p = (p / l).astype(bf)
            oT = jax.lax.dot_general(
                vT, p, (((1,), (1,)), ((), ())),
                preferred_element_type=f32)
            obuf[pl.ds(h * DH, DH), cols] = oT.astype(bf)

    pb = jax.lax.dot_general(
        obuf[...], wo_ref[...], (((0,), (0,)), ((), ())),
        preferred_element_type=f32)
    out_ref[...] = pb.astype(bf)

    pl.semaphore_wait(barrier, N_DEV - 1)

    rs = []
    for k in range(1, N_DEV):
        t = lax.rem(my + k, N_DEV)
        rdma = pltpu.make_async_remote_copy(
            src_ref=out_ref.at[pl.ds(t * CHUNK, CHUNK), :],
            dst_ref=comm_ref.at[N_DEV - 1 - k],
            send_sem=rs_send.at[k - 1],
            recv_sem=rs_recv.at[N_DEV - 1 - k],
            device_id=(t,),
            device_id_type=pltpu.DeviceIdType.MESH,
        )
        rdma.start()
        rs.append(rdma)

    sl_my = pl.ds(my * CHUNK, CHUNK)
    acc = out_ref[sl_my, :].astype(f32)
    for h in range(N_DEV - 2, -1, -1):
        recv_desc = pltpu.make_async_remote_copy(
            src_ref=comm_ref.at[h],
            dst_ref=comm_ref.at[h],
            send_sem=rs_recv.at[0],
            recv_sem=rs_recv.at[h],
            device_id=(my,),
            device_id_type=pltpu.DeviceIdType.MESH,
        )
        recv_desc.wait_recv()
        acc = acc + comm_ref[h].astype(f32)
    out_ref[sl_my, :] = acc.astype(bf)

    ag = []
    for k in range(1, N_DEV):
        t = lax.rem(my + k, N_DEV)
        rdma = pltpu.make_async_remote_copy(
            src_ref=out_ref.at[sl_my, :],
            dst_ref=out_ref.at[sl_my, :],
            send_sem=ag_send.at[k - 1],
            recv_sem=ag_recv.at[N_DEV - 1 - k],
            device_id=(t,),
            device_id_type=pltpu.DeviceIdType.MESH,
        )
        rdma.start()
        ag.append(rdma)

    for h in range(N_DEV - 1):
        recv_desc = pltpu.make_async_remote_copy(
            src_ref=comm_ref.at[h],
            dst_ref=out_ref.at[pl.ds(0, CHUNK), :],
            send_sem=ag_send.at[0],
            recv_sem=ag_recv.at[h],
            device_id=(my,),
            device_id_type=pltpu.DeviceIdType.MESH,
        )
        recv_desc.wait_recv()
    for r in rs:
        r.wait_send()
    for r in ag:
        r.wait_send()


def kernel(x, Wq, Wo, Wk, Wv):
    i = lax.axis_index("i")
    bf = jnp.bfloat16

    x2 = x.reshape(ROWS, D).astype(bf)
    wk_loc = lax.dynamic_slice_in_dim(Wk, i * KV_LOC * DH, KV_LOC * DH, 1)
    wv_loc = lax.dynamic_slice_in_dim(Wv, i * KV_LOC * DH, KV_LOC * DH, 1)

    out = pl.pallas_call(
        _body,
        out_shape=jax.ShapeDtypeStruct((ROWS, D), bf),
        in_specs=[pl.BlockSpec(memory_space=pltpu.VMEM)] * 5,
        out_specs=pl.BlockSpec(memory_space=pltpu.VMEM),
        scratch_shapes=[
            pltpu.VMEM((F, ROWS), bf),
            pltpu.VMEM((KV_LOC * DH, ROWS), bf),
            pltpu.VMEM((KV_LOC * DH, ROWS), bf),
            pltpu.VMEM((F, ROWS), bf),
            pltpu.VMEM((N_DEV - 1, CHUNK, D), bf),
            pltpu.SemaphoreType.DMA((N_DEV - 1,)),
            pltpu.SemaphoreType.DMA((N_DEV - 1,)),
            pltpu.SemaphoreType.DMA((N_DEV - 1,)),
            pltpu.SemaphoreType.DMA((N_DEV - 1,)),
        ],
        compiler_params=pltpu.CompilerParams(collective_id=0),
    )(x2, Wq.astype(bf), wk_loc.astype(bf), wv_loc.astype(bf), Wo.astype(bf))
    return out.reshape(B, SQ, D)
